# baseline (speedup 1.0000x reference)
import functools
import jax
import jax.numpy as jnp
from jax import lax
from jax.experimental import pallas as pl
from jax.experimental.pallas import tpu as pltpu
from jax.experimental.pallas import tpu_sc as plsc

N = 10000
E = 320000
D = 128
ED = 16
H = 256
G = 64

NC = 2
NS = 16
NW = NC * NS
EPW = E // NW
CB = 40
NCHUNK = EPW // CB
NPAD = 10240
RPT = NPAD // NS

NB = 10
NBR = NPAD // NB
EB = 160
EBR = E // EB

_LP0 = 9.97503255e-06
_LP1 = 9.99235484e-01
_LP2 = -4.90230723e-01
_LP3 = 2.85272681e-01
_LP4 = -1.31581825e-01
_LP5 = 3.04490045e-02


def _edge_tables_body(ea_ref, w_ref, t1_ref, t2_ref, t3_ref):
    ea = ea_ref[...]
    w = w_ref[...]
    t1_ref[...] = jnp.dot(ea, w[0], preferred_element_type=jnp.float32)
    t2_ref[...] = jnp.dot(ea, w[1], preferred_element_type=jnp.float32)
    t3_ref[...] = jnp.dot(ea, w[2], preferred_element_type=jnp.float32)


def _edge_tables(ea, we_stack):
    return pl.pallas_call(
        _edge_tables_body,
        grid=(EB,),
        in_specs=[
            pl.BlockSpec((EBR, ED), lambda i: (i, 0)),
            pl.BlockSpec((3, ED, 2 * D), lambda i: (0, 0, 0)),
        ],
        out_specs=[
            pl.BlockSpec((EBR, 2 * D), lambda i: (i, 0)),
            pl.BlockSpec((EBR, 2 * D), lambda i: (i, 0)),
            pl.BlockSpec((EBR, 2 * D), lambda i: (i, 0)),
        ],
        out_shape=[jax.ShapeDtypeStruct((E, 2 * D), jnp.float32)] * 3,
    )(ea, we_stack)


def _layer_tc_body(hp_ref, sa_ref, sb_ref, ca_ref, cb_ref, wd_ref, bd_ref,
                   ws_ref, h_ref, td_ref, ts_ref):
    cnt = jnp.maximum(ca_ref[...][:, 0] + cb_ref[...][:, 0], 1.0)
    h = hp_ref[...] + (sa_ref[...] + sb_ref[...]) / cnt[:, None]
    h_ref[...] = h
    td_ref[...] = (jnp.dot(h, wd_ref[...], preferred_element_type=jnp.float32)
                   + bd_ref[...])
    ts_ref[...] = jnp.dot(h, ws_ref[...], preferred_element_type=jnp.float32)


def _layer_tc(hp, sflat, cflat, wd, bd, ws):
    return pl.pallas_call(
        _layer_tc_body,
        grid=(NB,),
        in_specs=[
            pl.BlockSpec((NBR, D), lambda i: (i, 0)),
            pl.BlockSpec((NBR, D), lambda i: (i, 0)),
            pl.BlockSpec((NBR, D), lambda i: (i + NB, 0)),
            pl.BlockSpec((NBR, D), lambda i: (i, 0)),
            pl.BlockSpec((NBR, D), lambda i: (i + NB, 0)),
            pl.BlockSpec((D, 2 * D), lambda i: (0, 0)),
            pl.BlockSpec((1, 2 * D), lambda i: (0, 0)),
            pl.BlockSpec((D, 2 * D), lambda i: (0, 0)),
        ],
        out_specs=[
            pl.BlockSpec((NBR, D), lambda i: (i, 0)),
            pl.BlockSpec((NBR, 2 * D), lambda i: (i, 0)),
            pl.BlockSpec((NBR, 2 * D), lambda i: (i, 0)),
        ],
        out_shape=[
            jax.ShapeDtypeStruct((NPAD, D), jnp.float32),
            jax.ShapeDtypeStruct((NPAD, 2 * D), jnp.float32),
            jax.ShapeDtypeStruct((NPAD, 2 * D), jnp.float32),
        ],
    )(hp, sflat, sflat, cflat, cflat, wd, bd, ws)


def _final_tc_body(hp_ref, sa_ref, sb_ref, ca_ref, cb_ref, batch_ref,
                   w1_ref, b1_ref, w2_ref, b2_ref, out_ref, pool_acc, cnt_acc):
    i = pl.program_id(0)

    @pl.when(i == 0)
    def _():
        pool_acc[...] = jnp.zeros_like(pool_acc)
        cnt_acc[...] = jnp.zeros_like(cnt_acc)

    cnt = jnp.maximum(ca_ref[...][:, 0] + cb_ref[...][:, 0], 1.0)
    h = hp_ref[...] + (sa_ref[...] + sb_ref[...]) / cnt[:, None]
    ids = lax.broadcasted_iota(jnp.int32, (G, NBR), 0)
    oh = (batch_ref[0] == ids).astype(jnp.float32)
    pool_acc[...] += jnp.dot(oh, h, preferred_element_type=jnp.float32)
    cnt_acc[...] += jnp.broadcast_to(jnp.sum(oh, axis=1)[:, None], (G, D))

    @pl.when(i == NB - 1)
    def _():
        pooled = pool_acc[...] / jnp.maximum(cnt_acc[...], 1.0)
        hid = jnp.maximum(
            jnp.dot(pooled, w1_ref[...], preferred_element_type=jnp.float32)
            + b1_ref[...], 0.0)
        out_ref[...] = (jnp.dot(hid, w2_ref[...],
                                preferred_element_type=jnp.float32)
                        + b2_ref[...])


def _final_tc(hp, sflat, cflat, batch, w1, b1, w2, b2):
    return pl.pallas_call(
        _final_tc_body,
        grid=(NB,),
        in_specs=[
            pl.BlockSpec((NBR, D), lambda i: (i, 0)),
            pl.BlockSpec((NBR, D), lambda i: (i, 0)),
            pl.BlockSpec((NBR, D), lambda i: (i + NB, 0)),
            pl.BlockSpec((NBR, D), lambda i: (i, 0)),
            pl.BlockSpec((NBR, D), lambda i: (i + NB, 0)),
            pl.BlockSpec((1, 1, NBR), lambda i: (i, 0, 0)),
            pl.BlockSpec((D, H), lambda i: (0, 0)),
            pl.BlockSpec((1, H), lambda i: (0, 0)),
            pl.BlockSpec((H, 1), lambda i: (0, 0)),
            pl.BlockSpec((1, 1), lambda i: (0, 0)),
        ],
        out_specs=pl.BlockSpec((G, 1), lambda i: (0, 0)),
        out_shape=jax.ShapeDtypeStruct((G, 1), jnp.float32),
        scratch_shapes=[
            pltpu.VMEM((G, D), jnp.float32),
            pltpu.VMEM((G, D), jnp.float32),
        ],
    )(hp, sflat, sflat, cflat, cflat, batch, w1, b1, w2, b2)


def _softplus16(s):
    u = jnp.exp(jnp.minimum(s, -s))
    p = _LP5
    p = p * u + _LP4
    p = p * u + _LP3
    p = p * u + _LP2
    p = p * u + _LP1
    p = p * u + _LP0
    return jnp.maximum(s, 0.0) + p


def _sc_layer_body(td_hbm, ts_hbm, te_hbm, dst_hbm, src_hbm, zs_hbm,
                   s_out, dstv, srcv, gd, gs, tev, mv, acc, sem1, sem2):
    cid = lax.axis_index("c")
    sid = lax.axis_index("s")
    base = sid * RPT

    pltpu.sync_copy(zs_hbm.at[pl.ds(base, RPT)], acc.at[pl.ds(base, RPT)])

    plsc.subcore_barrier()

    wid = sid * NC + cid
    ebase = wid * EPW

    def chunk(k, carry):
        o = ebase + k * CB
        pltpu.sync_copy(dst_hbm.at[pl.ds(o, CB)], dstv)
        pltpu.sync_copy(src_hbm.at[pl.ds(o, CB)], srcv)
        cpd = pltpu.async_copy(td_hbm.at[dstv], gd, sem1)
        cps = pltpu.async_copy(ts_hbm.at[srcv], gs, sem2)
        pltpu.sync_copy(te_hbm.at[pl.ds(o, CB)], tev)
        cpd.wait()
        cps.wait()

        def edge(e, c2):
            for c8 in range(8):
                lo = c8 * 16
                hi = D + c8 * 16
                f = (gd[e, pl.ds(lo, 16)] + gs[e, pl.ds(lo, 16)]
                     + tev[e, pl.ds(lo, 16)])
                s = (gd[e, pl.ds(hi, 16)] + gs[e, pl.ds(hi, 16)]
                     + tev[e, pl.ds(hi, 16)])
                sig = 1.0 / (1.0 + jnp.exp(-f))
                mv[e, pl.ds(lo, 16)] = sig * _softplus16(s)
            return c2

        lax.fori_loop(0, CB, edge, 0, unroll=False)
        pltpu.sync_copy(mv, acc.at[dstv], add=True)
        return carry

    lax.fori_loop(0, NCHUNK, chunk, 0, unroll=False)
    plsc.subcore_barrier()

    pltpu.sync_copy(acc.at[pl.ds(base, RPT)],
                    s_out.at[pl.ds(cid * NPAD + base, RPT)])


def _sc_count_body(dst_hbm, zc_hbm, c_out, dstv, onesv, cacc):
    cid = lax.axis_index("c")
    sid = lax.axis_index("s")
    base = sid * RPT

    pltpu.sync_copy(zc_hbm.at[pl.ds(base, RPT)], cacc.at[pl.ds(base, RPT)])
    for r in range(CB):
        for c8 in range(8):
            onesv[r, pl.ds(c8 * 16, 16)] = jnp.ones((16,), jnp.float32)

    plsc.subcore_barrier()

    wid = sid * NC + cid
    ebase = wid * EPW

    def chunk(k, carry):
        o = ebase + k * CB
        pltpu.sync_copy(dst_hbm.at[pl.ds(o, CB)], dstv)
        pltpu.sync_copy(onesv, cacc.at[dstv], add=True)
        return carry

    lax.fori_loop(0, NCHUNK, chunk, 0, unroll=False)
    plsc.subcore_barrier()

    pltpu.sync_copy(cacc.at[pl.ds(base, RPT)],
                    c_out.at[pl.ds(cid * NPAD + base, RPT)])


@functools.lru_cache(maxsize=1)
def _make_sc_count():
    return pl.kernel(
        _sc_count_body,
        out_type=jax.ShapeDtypeStruct((NC * NPAD, D), jnp.float32),
        mesh=plsc.VectorSubcoreMesh(core_axis_name="c", subcore_axis_name="s",
                                    num_cores=NC, num_subcores=NS),
        scratch_types=[
            pltpu.VMEM((CB,), jnp.int32),
            pltpu.VMEM((CB, D), jnp.float32),
            pltpu.VMEM_SHARED((NPAD, D), jnp.float32),
        ],
    )


@functools.lru_cache(maxsize=1)
def _make_sc_layer():
    return pl.kernel(
        _sc_layer_body,
        out_type=jax.ShapeDtypeStruct((NC * NPAD, D), jnp.float32),
        mesh=plsc.VectorSubcoreMesh(core_axis_name="c", subcore_axis_name="s",
                                    num_cores=NC, num_subcores=NS),
        scratch_types=[
            pltpu.VMEM((CB,), jnp.int32),
            pltpu.VMEM((CB,), jnp.int32),
            pltpu.VMEM((CB, 2 * D), jnp.float32),
            pltpu.VMEM((CB, 2 * D), jnp.float32),
            pltpu.VMEM((CB, 2 * D), jnp.float32),
            pltpu.VMEM((CB, D), jnp.float32),
            pltpu.VMEM_SHARED((NPAD, D), jnp.float32),
            pltpu.SemaphoreType.DMA,
            pltpu.SemaphoreType.DMA,
        ],
    )


def _sc_layer(*args):
    return _make_sc_layer()(*args)


def kernel(x, edge_index, edge_attr, batch, Wf1, bf1, Ws1, bs1, Wf2, bf2,
           Ws2, bs2, Wf3, bf3, Ws3, bs3, W1, b1, W2, b2):
    f32 = jnp.float32
    src = edge_index[0].astype(jnp.int32)
    dst = edge_index[1].astype(jnp.int32)
    batch = batch.astype(jnp.int32)

    def packs(Wf, bf, Ws, bs):
        wd = jnp.concatenate([Wf[:D], Ws[:D]], axis=1)
        wsrc = jnp.concatenate([Wf[D:2 * D], Ws[D:2 * D]], axis=1)
        we = jnp.concatenate([Wf[2 * D:], Ws[2 * D:]], axis=1)
        bd = jnp.concatenate([bf, bs]).reshape(1, 2 * D)
        return wd, wsrc, we, bd

    wd1, wsrc1, we1, bd1 = packs(Wf1, bf1, Ws1, bs1)
    wd2, wsrc2, we2, bd2 = packs(Wf2, bf2, Ws2, bs2)
    wd3, wsrc3, we3, bd3 = packs(Wf3, bf3, Ws3, bs3)
    we_stack = jnp.stack([we1, we2, we3])

    te1, te2, te3 = _edge_tables(edge_attr.astype(f32), we_stack)

    xp = jnp.zeros((NPAD, D), f32).at[:N].set(x.astype(f32))
    bp = jnp.full((NPAD,), G, jnp.int32).at[:N].set(batch)

    zs1 = jnp.zeros((NPAD, D), f32)
    zsf = jnp.zeros((NC * NPAD, D), f32)

    cf = _make_sc_count()(dst, zs1)

    _, td1, ts1 = _layer_tc(xp, zsf, cf, wd1, bd1, wsrc1)
    sf1 = _sc_layer(td1, ts1, te1, dst, src, zs1)
    h1, td2, ts2 = _layer_tc(xp, sf1, cf, wd2, bd2, wsrc2)
    sf2 = _sc_layer(td2, ts2, te2, dst, src, zs1)
    h2, td3, ts3 = _layer_tc(h1, sf2, cf, wd3, bd3, wsrc3)
    sf3 = _sc_layer(td3, ts3, te3, dst, src, zs1)

    return _final_tc(h2, sf3, cf, bp.reshape(NB, 1, NBR), W1,
                     b1.reshape(1, H), W2, b2.reshape(1, 1))


# speedup vs baseline: 2.8887x; 2.8887x over previous
import functools
import jax
import jax.numpy as jnp
from jax import lax
from jax.experimental import pallas as pl
from jax.experimental.pallas import tpu as pltpu
from jax.experimental.pallas import tpu_sc as plsc

N = 10000
E = 320000
D = 128
ED = 16
H = 256
G = 64

NC = 2
NS = 16
NW = NC * NS
EPW = E // NW
CB = 80
NCHUNK = EPW // CB
NPAD = 10240
RPT = NPAD // NS

NB = 10
NBR = NPAD // NB
EB = 160
EBR = E // EB


def _layer_tc_body(hp_ref, sa_ref, sb_ref, ca_ref, cb_ref, wd_ref, bd_ref,
                   ws_ref, h_ref, td_ref, ts_ref):
    cnt = jnp.maximum(ca_ref[...][:, 0] + cb_ref[...][:, 0], 1.0)
    h = hp_ref[...] + (sa_ref[...] + sb_ref[...]) / cnt[:, None]
    h_ref[...] = h
    td_ref[...] = (jnp.dot(h, wd_ref[...], preferred_element_type=jnp.float32)
                   + bd_ref[...])
    ts_ref[...] = jnp.dot(h, ws_ref[...], preferred_element_type=jnp.float32)


def _layer_tc(hp, sflat, cflat, wd, bd, ws):
    return pl.pallas_call(
        _layer_tc_body,
        grid=(NB,),
        in_specs=[
            pl.BlockSpec((NBR, D), lambda i: (i, 0)),
            pl.BlockSpec((NBR, D), lambda i: (i, 0)),
            pl.BlockSpec((NBR, D), lambda i: (i + NB, 0)),
            pl.BlockSpec((NBR, D), lambda i: (i, 0)),
            pl.BlockSpec((NBR, D), lambda i: (i + NB, 0)),
            pl.BlockSpec((D, 2 * D), lambda i: (0, 0)),
            pl.BlockSpec((1, 2 * D), lambda i: (0, 0)),
            pl.BlockSpec((D, 2 * D), lambda i: (0, 0)),
        ],
        out_specs=[
            pl.BlockSpec((NBR, D), lambda i: (i, 0)),
            pl.BlockSpec((NBR, 2 * D), lambda i: (i, 0)),
            pl.BlockSpec((NBR, 2 * D), lambda i: (i, 0)),
        ],
        out_shape=[
            jax.ShapeDtypeStruct((NPAD, D), jnp.float32),
            jax.ShapeDtypeStruct((NPAD, 2 * D), jnp.float32),
            jax.ShapeDtypeStruct((NPAD, 2 * D), jnp.float32),
        ],
    )(hp, sflat, sflat, cflat, cflat, wd, bd, ws)


def _msg_tc_body(tdg_ref, tsg_ref, ea_ref, we_ref, m_ref):
    z = (tdg_ref[...] + tsg_ref[...]
         + jnp.dot(ea_ref[...], we_ref[...],
                   preferred_element_type=jnp.float32))
    f = z[:, :D]
    s = z[:, D:]
    sig = 1.0 / (1.0 + jnp.exp(-f))
    sp = jnp.maximum(s, 0.0) + jnp.log1p(jnp.exp(-jnp.abs(s)))
    m_ref[...] = sig * sp


def _msg_tc(tdg, tsg, ea, we):
    return pl.pallas_call(
        _msg_tc_body,
        grid=(EB,),
        in_specs=[
            pl.BlockSpec((EBR, 2 * D), lambda i: (i, 0)),
            pl.BlockSpec((EBR, 2 * D), lambda i: (i, 0)),
            pl.BlockSpec((EBR, ED), lambda i: (i, 0)),
            pl.BlockSpec((ED, 2 * D), lambda i: (0, 0)),
        ],
        out_specs=pl.BlockSpec((EBR, D), lambda i: (i, 0)),
        out_shape=jax.ShapeDtypeStruct((E, D), jnp.float32),
    )(tdg, tsg, ea, we)


def _final_tc_body(hp_ref, sa_ref, sb_ref, ca_ref, cb_ref, batch_ref,
                   w1_ref, b1_ref, w2_ref, b2_ref, out_ref, pool_acc, cnt_acc):
    i = pl.program_id(0)

    @pl.when(i == 0)
    def _():
        pool_acc[...] = jnp.zeros_like(pool_acc)
        cnt_acc[...] = jnp.zeros_like(cnt_acc)

    cnt = jnp.maximum(ca_ref[...][:, 0] + cb_ref[...][:, 0], 1.0)
    h = hp_ref[...] + (sa_ref[...] + sb_ref[...]) / cnt[:, None]
    ids = lax.broadcasted_iota(jnp.int32, (G, NBR), 0)
    oh = (batch_ref[0] == ids).astype(jnp.float32)
    pool_acc[...] += jnp.dot(oh, h, preferred_element_type=jnp.float32)
    cnt_acc[...] += jnp.broadcast_to(jnp.sum(oh, axis=1)[:, None], (G, D))

    @pl.when(i == NB - 1)
    def _():
        pooled = pool_acc[...] / jnp.maximum(cnt_acc[...], 1.0)
        hid = jnp.maximum(
            jnp.dot(pooled, w1_ref[...], preferred_element_type=jnp.float32)
            + b1_ref[...], 0.0)
        out_ref[...] = (jnp.dot(hid, w2_ref[...],
                                preferred_element_type=jnp.float32)
                        + b2_ref[...])


def _final_tc(hp, sflat, cflat, batch, w1, b1, w2, b2):
    return pl.pallas_call(
        _final_tc_body,
        grid=(NB,),
        in_specs=[
            pl.BlockSpec((NBR, D), lambda i: (i, 0)),
            pl.BlockSpec((NBR, D), lambda i: (i, 0)),
            pl.BlockSpec((NBR, D), lambda i: (i + NB, 0)),
            pl.BlockSpec((NBR, D), lambda i: (i, 0)),
            pl.BlockSpec((NBR, D), lambda i: (i + NB, 0)),
            pl.BlockSpec((1, 1, NBR), lambda i: (i, 0, 0)),
            pl.BlockSpec((D, H), lambda i: (0, 0)),
            pl.BlockSpec((1, H), lambda i: (0, 0)),
            pl.BlockSpec((H, 1), lambda i: (0, 0)),
            pl.BlockSpec((1, 1), lambda i: (0, 0)),
        ],
        out_specs=pl.BlockSpec((G, 1), lambda i: (0, 0)),
        out_shape=jax.ShapeDtypeStruct((G, 1), jnp.float32),
        scratch_shapes=[
            pltpu.VMEM((G, D), jnp.float32),
            pltpu.VMEM((G, D), jnp.float32),
        ],
    )(hp, sflat, sflat, cflat, cflat, batch, w1, b1, w2, b2)


def _sc_gather_body(td_hbm, ts_hbm, dst_hbm, src_hbm, tdg_out, tsg_out,
                    dstv, srcv, gd, gs, sem1, sem2):
    cid = lax.axis_index("c")
    sid = lax.axis_index("s")
    wid = sid * NC + cid
    ebase = wid * EPW

    def chunk(k, carry):
        o = ebase + k * CB
        pltpu.sync_copy(dst_hbm.at[pl.ds(o, CB)], dstv)
        pltpu.sync_copy(src_hbm.at[pl.ds(o, CB)], srcv)
        cpd = pltpu.async_copy(td_hbm.at[dstv], gd, sem1)
        cps = pltpu.async_copy(ts_hbm.at[srcv], gs, sem2)
        cpd.wait()
        pltpu.sync_copy(gd, tdg_out.at[pl.ds(o, CB)])
        cps.wait()
        pltpu.sync_copy(gs, tsg_out.at[pl.ds(o, CB)])
        return carry

    lax.fori_loop(0, NCHUNK, chunk, 0, unroll=False)


@functools.lru_cache(maxsize=1)
def _make_sc_gather():
    return pl.kernel(
        _sc_gather_body,
        out_type=[
            jax.ShapeDtypeStruct((E, 2 * D), jnp.float32),
            jax.ShapeDtypeStruct((E, 2 * D), jnp.float32),
        ],
        mesh=plsc.VectorSubcoreMesh(core_axis_name="c", subcore_axis_name="s",
                                    num_cores=NC, num_subcores=NS),
        scratch_types=[
            pltpu.VMEM((CB,), jnp.int32),
            pltpu.VMEM((CB,), jnp.int32),
            pltpu.VMEM((CB, 2 * D), jnp.float32),
            pltpu.VMEM((CB, 2 * D), jnp.float32),
            pltpu.SemaphoreType.DMA,
            pltpu.SemaphoreType.DMA,
        ],
    )


def _sc_scatter_body(m_hbm, dst_hbm, zs_hbm, s_out, dstv, mv, acc, sem):
    cid = lax.axis_index("c")
    sid = lax.axis_index("s")
    base = sid * RPT

    pltpu.sync_copy(zs_hbm.at[pl.ds(base, RPT)], acc.at[pl.ds(base, RPT)])
    plsc.subcore_barrier()

    wid = sid * NC + cid
    ebase = wid * EPW

    def chunk(k, carry):
        o = ebase + k * CB
        pltpu.sync_copy(dst_hbm.at[pl.ds(o, CB)], dstv)
        pltpu.sync_copy(m_hbm.at[pl.ds(o, CB)], mv)
        pltpu.sync_copy(mv, acc.at[dstv], add=True)
        return carry

    lax.fori_loop(0, NCHUNK, chunk, 0, unroll=False)
    plsc.subcore_barrier()

    pltpu.sync_copy(acc.at[pl.ds(base, RPT)],
                    s_out.at[pl.ds(cid * NPAD + base, RPT)])


@functools.lru_cache(maxsize=1)
def _make_sc_scatter():
    return pl.kernel(
        _sc_scatter_body,
        out_type=jax.ShapeDtypeStruct((NC * NPAD, D), jnp.float32),
        mesh=plsc.VectorSubcoreMesh(core_axis_name="c", subcore_axis_name="s",
                                    num_cores=NC, num_subcores=NS),
        scratch_types=[
            pltpu.VMEM((CB,), jnp.int32),
            pltpu.VMEM((CB, D), jnp.float32),
            pltpu.VMEM_SHARED((NPAD, D), jnp.float32),
            pltpu.SemaphoreType.DMA,
        ],
    )


def _sc_count_body(dst_hbm, zc_hbm, c_out, dstv, onesv, cacc):
    cid = lax.axis_index("c")
    sid = lax.axis_index("s")
    base = sid * RPT

    pltpu.sync_copy(zc_hbm.at[pl.ds(base, RPT)], cacc.at[pl.ds(base, RPT)])
    for r in range(CB):
        for c8 in range(8):
            onesv[r, pl.ds(c8 * 16, 16)] = jnp.ones((16,), jnp.float32)

    plsc.subcore_barrier()

    wid = sid * NC + cid
    ebase = wid * EPW

    def chunk(k, carry):
        o = ebase + k * CB
        pltpu.sync_copy(dst_hbm.at[pl.ds(o, CB)], dstv)
        pltpu.sync_copy(onesv, cacc.at[dstv], add=True)
        return carry

    lax.fori_loop(0, NCHUNK, chunk, 0, unroll=False)
    plsc.subcore_barrier()

    pltpu.sync_copy(cacc.at[pl.ds(base, RPT)],
                    c_out.at[pl.ds(cid * NPAD + base, RPT)])


@functools.lru_cache(maxsize=1)
def _make_sc_count():
    return pl.kernel(
        _sc_count_body,
        out_type=jax.ShapeDtypeStruct((NC * NPAD, D), jnp.float32),
        mesh=plsc.VectorSubcoreMesh(core_axis_name="c", subcore_axis_name="s",
                                    num_cores=NC, num_subcores=NS),
        scratch_types=[
            pltpu.VMEM((CB,), jnp.int32),
            pltpu.VMEM((CB, D), jnp.float32),
            pltpu.VMEM_SHARED((NPAD, D), jnp.float32),
        ],
    )


def kernel(x, edge_index, edge_attr, batch, Wf1, bf1, Ws1, bs1, Wf2, bf2,
           Ws2, bs2, Wf3, bf3, Ws3, bs3, W1, b1, W2, b2):
    f32 = jnp.float32
    src = edge_index[0].astype(jnp.int32)
    dst = edge_index[1].astype(jnp.int32)
    batch = batch.astype(jnp.int32)
    ea = edge_attr.astype(f32)

    def packs(Wf, bf, Ws, bs):
        wd = jnp.concatenate([Wf[:D], Ws[:D]], axis=1)
        wsrc = jnp.concatenate([Wf[D:2 * D], Ws[D:2 * D]], axis=1)
        we = jnp.concatenate([Wf[2 * D:], Ws[2 * D:]], axis=1)
        bd = jnp.concatenate([bf, bs]).reshape(1, 2 * D)
        return wd, wsrc, we, bd

    wd1, wsrc1, we1, bd1 = packs(Wf1, bf1, Ws1, bs1)
    wd2, wsrc2, we2, bd2 = packs(Wf2, bf2, Ws2, bs2)
    wd3, wsrc3, we3, bd3 = packs(Wf3, bf3, Ws3, bs3)

    xp = jnp.zeros((NPAD, D), f32).at[:N].set(x.astype(f32))
    bp = jnp.full((NPAD,), G, jnp.int32).at[:N].set(batch)

    zs1 = jnp.zeros((NPAD, D), f32)
    zsf = jnp.zeros((NC * NPAD, D), f32)

    cf = _make_sc_count()(dst, zs1)
    gather = _make_sc_gather()
    scatter = _make_sc_scatter()

    def layer(hprev, sflat, wd, bd, wsrc, we):
        h, td, ts = _layer_tc(hprev, sflat, cf, wd, bd, wsrc)
        tdg, tsg = gather(td, ts, dst, src)
        m = _msg_tc(tdg, tsg, ea, we)
        return h, scatter(m, dst, zs1)

    _, sf1 = layer(xp, zsf, wd1, bd1, wsrc1, we1)
    h1, sf2 = layer(xp, sf1, wd2, bd2, wsrc2, we2)
    h2, sf3 = layer(h1, sf2, wd3, bd3, wsrc3, we3)

    return _final_tc(h2, sf3, cf, bp.reshape(NB, 1, NBR), W1,
                     b1.reshape(1, H), W2, b2.reshape(1, 1))


# speedup vs baseline: 4.6443x; 1.6078x over previous
import functools
import jax
import jax.numpy as jnp
from jax import lax
from jax.experimental import pallas as pl
from jax.experimental.pallas import tpu as pltpu
from jax.experimental.pallas import tpu_sc as plsc

N = 10000
E = 320000
D = 128
ED = 16
H = 256
G = 64

NC = 2
NS = 16
NW = NC * NS
EPW = E // NW
CB = 80
NCHUNK = EPW // CB
NPAIR = NCHUNK // 2
NPAD = 10240
RPT = NPAD // NS

NB = 10
NBR = NPAD // NB
EB = 160
EBR = E // EB


def _resid_tc_body(hp_ref, sa_ref, sb_ref, ca_ref, cb_ref, h_ref):
    cnt = jnp.maximum(ca_ref[...][:, 0] + cb_ref[...][:, 0], 1.0)
    h_ref[...] = hp_ref[...] + (sa_ref[...] + sb_ref[...]) / cnt[:, None]


def _resid_tc(hp, sflat, cflat):
    return pl.pallas_call(
        _resid_tc_body,
        grid=(NB,),
        in_specs=[
            pl.BlockSpec((NBR, D), lambda i: (i, 0)),
            pl.BlockSpec((NBR, D), lambda i: (i, 0)),
            pl.BlockSpec((NBR, D), lambda i: (i + NB, 0)),
            pl.BlockSpec((NBR, D), lambda i: (i, 0)),
            pl.BlockSpec((NBR, D), lambda i: (i + NB, 0)),
        ],
        out_specs=pl.BlockSpec((NBR, D), lambda i: (i, 0)),
        out_shape=jax.ShapeDtypeStruct((NPAD, D), jnp.float32),
    )(hp, sflat, sflat, cflat, cflat)


def _msg_tc_body(hd_ref, hs_ref, ea_ref, wd_ref, ws_ref, we_ref, bd_ref,
                 m_ref):
    z = (jnp.dot(hd_ref[...], wd_ref[...], preferred_element_type=jnp.float32)
         + jnp.dot(hs_ref[...], ws_ref[...],
                   preferred_element_type=jnp.float32)
         + jnp.dot(ea_ref[...], we_ref[...],
                   preferred_element_type=jnp.float32)
         + bd_ref[...])
    f = z[:, :D]
    s = z[:, D:]
    sig = 1.0 / (1.0 + jnp.exp(-f))
    sp = jnp.maximum(s, 0.0) + jnp.log1p(jnp.exp(-jnp.abs(s)))
    m_ref[...] = sig * sp


def _msg_tc(hd, hs, ea, wd, ws, we, bd):
    return pl.pallas_call(
        _msg_tc_body,
        grid=(EB,),
        in_specs=[
            pl.BlockSpec((EBR, D), lambda i: (i, 0)),
            pl.BlockSpec((EBR, D), lambda i: (i, 0)),
            pl.BlockSpec((EBR, ED), lambda i: (i, 0)),
            pl.BlockSpec((D, 2 * D), lambda i: (0, 0)),
            pl.BlockSpec((D, 2 * D), lambda i: (0, 0)),
            pl.BlockSpec((ED, 2 * D), lambda i: (0, 0)),
            pl.BlockSpec((1, 2 * D), lambda i: (0, 0)),
        ],
        out_specs=pl.BlockSpec((EBR, D), lambda i: (i, 0)),
        out_shape=jax.ShapeDtypeStruct((E, D), jnp.float32),
    )(hd, hs, ea, wd, ws, we, bd)


def _final_tc_body(hp_ref, sa_ref, sb_ref, ca_ref, cb_ref, batch_ref,
                   w1_ref, b1_ref, w2_ref, b2_ref, out_ref, pool_acc, cnt_acc):
    i = pl.program_id(0)

    @pl.when(i == 0)
    def _():
        pool_acc[...] = jnp.zeros_like(pool_acc)
        cnt_acc[...] = jnp.zeros_like(cnt_acc)

    cnt = jnp.maximum(ca_ref[...][:, 0] + cb_ref[...][:, 0], 1.0)
    h = hp_ref[...] + (sa_ref[...] + sb_ref[...]) / cnt[:, None]
    ids = lax.broadcasted_iota(jnp.int32, (G, NBR), 0)
    oh = (batch_ref[0] == ids).astype(jnp.float32)
    pool_acc[...] += jnp.dot(oh, h, preferred_element_type=jnp.float32)
    cnt_acc[...] += jnp.broadcast_to(jnp.sum(oh, axis=1)[:, None], (G, D))

    @pl.when(i == NB - 1)
    def _():
        pooled = pool_acc[...] / jnp.maximum(cnt_acc[...], 1.0)
        hid = jnp.maximum(
            jnp.dot(pooled, w1_ref[...], preferred_element_type=jnp.float32)
            + b1_ref[...], 0.0)
        out_ref[...] = (jnp.dot(hid, w2_ref[...],
                                preferred_element_type=jnp.float32)
                        + b2_ref[...])


def _final_tc(hp, sflat, cflat, batch, w1, b1, w2, b2):
    return pl.pallas_call(
        _final_tc_body,
        grid=(NB,),
        in_specs=[
            pl.BlockSpec((NBR, D), lambda i: (i, 0)),
            pl.BlockSpec((NBR, D), lambda i: (i, 0)),
            pl.BlockSpec((NBR, D), lambda i: (i + NB, 0)),
            pl.BlockSpec((NBR, D), lambda i: (i, 0)),
            pl.BlockSpec((NBR, D), lambda i: (i + NB, 0)),
            pl.BlockSpec((1, 1, NBR), lambda i: (i, 0, 0)),
            pl.BlockSpec((D, H), lambda i: (0, 0)),
            pl.BlockSpec((1, H), lambda i: (0, 0)),
            pl.BlockSpec((H, 1), lambda i: (0, 0)),
            pl.BlockSpec((1, 1), lambda i: (0, 0)),
        ],
        out_specs=pl.BlockSpec((G, 1), lambda i: (0, 0)),
        out_shape=jax.ShapeDtypeStruct((G, 1), jnp.float32),
        scratch_shapes=[
            pltpu.VMEM((G, D), jnp.float32),
            pltpu.VMEM((G, D), jnp.float32),
        ],
    )(hp, sflat, sflat, cflat, cflat, batch, w1, b1, w2, b2)


def _sc_gather_body(h_hbm, dst_hbm, src_hbm, hd_out, hs_out,
                    d0, s0, d1, s1, gA, gB, gC, gD,
                    semA, semB, semC, semD, semWA, semWB, semWC, semWD):
    cid = lax.axis_index("c")
    sid = lax.axis_index("s")
    wid = sid * NC + cid
    ebase = wid * EPW

    def pair(j, carry):
        o0 = ebase + (2 * j) * CB
        o1 = o0 + CB
        pltpu.sync_copy(dst_hbm.at[pl.ds(o0, CB)], d0)
        pltpu.sync_copy(src_hbm.at[pl.ds(o0, CB)], s0)
        cpA = pltpu.async_copy(h_hbm.at[d0], gA, semA)
        cpB = pltpu.async_copy(h_hbm.at[s0], gB, semB)
        pltpu.sync_copy(dst_hbm.at[pl.ds(o1, CB)], d1)
        pltpu.sync_copy(src_hbm.at[pl.ds(o1, CB)], s1)
        cpC = pltpu.async_copy(h_hbm.at[d1], gC, semC)
        cpD = pltpu.async_copy(h_hbm.at[s1], gD, semD)
        cpA.wait()
        wA = pltpu.async_copy(gA, hd_out.at[pl.ds(o0, CB)], semWA)
        cpB.wait()
        wB = pltpu.async_copy(gB, hs_out.at[pl.ds(o0, CB)], semWB)
        cpC.wait()
        wC = pltpu.async_copy(gC, hd_out.at[pl.ds(o1, CB)], semWC)
        cpD.wait()
        wD = pltpu.async_copy(gD, hs_out.at[pl.ds(o1, CB)], semWD)
        wA.wait()
        wB.wait()
        wC.wait()
        wD.wait()
        return carry

    lax.fori_loop(0, NPAIR, pair, 0, unroll=False)

    o = ebase + (NCHUNK - 1) * CB
    pltpu.sync_copy(dst_hbm.at[pl.ds(o, CB)], d0)
    pltpu.sync_copy(src_hbm.at[pl.ds(o, CB)], s0)
    cpA = pltpu.async_copy(h_hbm.at[d0], gA, semA)
    cpB = pltpu.async_copy(h_hbm.at[s0], gB, semB)
    cpA.wait()
    pltpu.sync_copy(gA, hd_out.at[pl.ds(o, CB)])
    cpB.wait()
    pltpu.sync_copy(gB, hs_out.at[pl.ds(o, CB)])


@functools.lru_cache(maxsize=1)
def _make_sc_gather():
    return pl.kernel(
        _sc_gather_body,
        out_type=[
            jax.ShapeDtypeStruct((E, D), jnp.float32),
            jax.ShapeDtypeStruct((E, D), jnp.float32),
        ],
        mesh=plsc.VectorSubcoreMesh(core_axis_name="c", subcore_axis_name="s",
                                    num_cores=NC, num_subcores=NS),
        scratch_types=[
            pltpu.VMEM((CB,), jnp.int32),
            pltpu.VMEM((CB,), jnp.int32),
            pltpu.VMEM((CB,), jnp.int32),
            pltpu.VMEM((CB,), jnp.int32),
            pltpu.VMEM((CB, D), jnp.float32),
            pltpu.VMEM((CB, D), jnp.float32),
            pltpu.VMEM((CB, D), jnp.float32),
            pltpu.VMEM((CB, D), jnp.float32),
            pltpu.SemaphoreType.DMA,
            pltpu.SemaphoreType.DMA,
            pltpu.SemaphoreType.DMA,
            pltpu.SemaphoreType.DMA,
            pltpu.SemaphoreType.DMA,
            pltpu.SemaphoreType.DMA,
            pltpu.SemaphoreType.DMA,
            pltpu.SemaphoreType.DMA,
        ],
    )


def _sc_scatter_body(m_hbm, dst_hbm, zs_hbm, s_out, d0, d1, m0, m1,
                     acc, semA, semB):
    cid = lax.axis_index("c")
    sid = lax.axis_index("s")
    base = sid * RPT

    pltpu.sync_copy(zs_hbm.at[pl.ds(base, RPT)], acc.at[pl.ds(base, RPT)])
    plsc.subcore_barrier()

    wid = sid * NC + cid
    ebase = wid * EPW

    def pair(j, carry):
        o0 = ebase + (2 * j) * CB
        o1 = o0 + CB
        pltpu.sync_copy(dst_hbm.at[pl.ds(o0, CB)], d0)
        cpA = pltpu.async_copy(m_hbm.at[pl.ds(o0, CB)], m0, semA)
        pltpu.sync_copy(dst_hbm.at[pl.ds(o1, CB)], d1)
        cpB = pltpu.async_copy(m_hbm.at[pl.ds(o1, CB)], m1, semB)
        cpA.wait()
        pltpu.sync_copy(m0, acc.at[d0], add=True)
        cpB.wait()
        pltpu.sync_copy(m1, acc.at[d1], add=True)
        return carry

    lax.fori_loop(0, NPAIR, pair, 0, unroll=False)

    o = ebase + (NCHUNK - 1) * CB
    pltpu.sync_copy(dst_hbm.at[pl.ds(o, CB)], d0)
    pltpu.sync_copy(m_hbm.at[pl.ds(o, CB)], m0)
    pltpu.sync_copy(m0, acc.at[d0], add=True)

    plsc.subcore_barrier()

    pltpu.sync_copy(acc.at[pl.ds(base, RPT)],
                    s_out.at[pl.ds(cid * NPAD + base, RPT)])


@functools.lru_cache(maxsize=1)
def _make_sc_scatter():
    return pl.kernel(
        _sc_scatter_body,
        out_type=jax.ShapeDtypeStruct((NC * NPAD, D), jnp.float32),
        mesh=plsc.VectorSubcoreMesh(core_axis_name="c", subcore_axis_name="s",
                                    num_cores=NC, num_subcores=NS),
        scratch_types=[
            pltpu.VMEM((CB,), jnp.int32),
            pltpu.VMEM((CB,), jnp.int32),
            pltpu.VMEM((CB, D), jnp.float32),
            pltpu.VMEM((CB, D), jnp.float32),
            pltpu.VMEM_SHARED((NPAD, D), jnp.float32),
            pltpu.SemaphoreType.DMA,
            pltpu.SemaphoreType.DMA,
        ],
    )


def _sc_count_body(dst_hbm, zc_hbm, c_out, dstv, onesv, cacc):
    cid = lax.axis_index("c")
    sid = lax.axis_index("s")
    base = sid * RPT

    pltpu.sync_copy(zc_hbm.at[pl.ds(base, RPT)], cacc.at[pl.ds(base, RPT)])
    for r in range(CB):
        for c8 in range(8):
            onesv[r, pl.ds(c8 * 16, 16)] = jnp.ones((16,), jnp.float32)

    plsc.subcore_barrier()

    wid = sid * NC + cid
    ebase = wid * EPW

    def chunk(k, carry):
        o = ebase + k * CB
        pltpu.sync_copy(dst_hbm.at[pl.ds(o, CB)], dstv)
        pltpu.sync_copy(onesv, cacc.at[dstv], add=True)
        return carry

    lax.fori_loop(0, NCHUNK, chunk, 0, unroll=False)
    plsc.subcore_barrier()

    pltpu.sync_copy(cacc.at[pl.ds(base, RPT)],
                    c_out.at[pl.ds(cid * NPAD + base, RPT)])


@functools.lru_cache(maxsize=1)
def _make_sc_count():
    return pl.kernel(
        _sc_count_body,
        out_type=jax.ShapeDtypeStruct((NC * NPAD, D), jnp.float32),
        mesh=plsc.VectorSubcoreMesh(core_axis_name="c", subcore_axis_name="s",
                                    num_cores=NC, num_subcores=NS),
        scratch_types=[
            pltpu.VMEM((CB,), jnp.int32),
            pltpu.VMEM((CB, D), jnp.float32),
            pltpu.VMEM_SHARED((NPAD, D), jnp.float32),
        ],
    )


def kernel(x, edge_index, edge_attr, batch, Wf1, bf1, Ws1, bs1, Wf2, bf2,
           Ws2, bs2, Wf3, bf3, Ws3, bs3, W1, b1, W2, b2):
    f32 = jnp.float32
    src = edge_index[0].astype(jnp.int32)
    dst = edge_index[1].astype(jnp.int32)
    batch = batch.astype(jnp.int32)
    ea = edge_attr.astype(f32)

    def packs(Wf, bf, Ws, bs):
        wd = jnp.concatenate([Wf[:D], Ws[:D]], axis=1)
        wsrc = jnp.concatenate([Wf[D:2 * D], Ws[D:2 * D]], axis=1)
        we = jnp.concatenate([Wf[2 * D:], Ws[2 * D:]], axis=1)
        bd = jnp.concatenate([bf, bs]).reshape(1, 2 * D)
        return wd, wsrc, we, bd

    wd1, wsrc1, we1, bd1 = packs(Wf1, bf1, Ws1, bs1)
    wd2, wsrc2, we2, bd2 = packs(Wf2, bf2, Ws2, bs2)
    wd3, wsrc3, we3, bd3 = packs(Wf3, bf3, Ws3, bs3)

    xp = jnp.zeros((NPAD, D), f32).at[:N].set(x.astype(f32))
    bp = jnp.full((NPAD,), G, jnp.int32).at[:N].set(batch)
    zs1 = jnp.zeros((NPAD, D), f32)

    cf = _make_sc_count()(dst, zs1)
    gather = _make_sc_gather()
    scatter = _make_sc_scatter()

    def layer(h, wd, wsrc, we, bd):
        hd, hs = gather(h, dst, src)
        m = _msg_tc(hd, hs, ea, wd, wsrc, we, bd)
        return scatter(m, dst, zs1)

    sf1 = layer(xp, wd1, wsrc1, we1, bd1)
    h1 = _resid_tc(xp, sf1, cf)
    sf2 = layer(h1, wd2, wsrc2, we2, bd2)
    h2 = _resid_tc(h1, sf2, cf)
    sf3 = layer(h2, wd3, wsrc3, we3, bd3)

    return _final_tc(h2, sf3, cf, bp.reshape(NB, 1, NBR), W1,
                     b1.reshape(1, H), W2, b2.reshape(1, 1))
